# baseline (speedup 1.0000x reference)
import functools

import jax
import jax.numpy as jnp
from jax import lax
from jax.experimental import pallas as pl
from jax.experimental.pallas import tpu as pltpu
from jax.experimental.pallas import tpu_sc as plsc

VOCAB = 110000
EMB = 64
H = 128
NCLS = 20
B = 1024
L = 200

TOTAL = B * L
NW = 32
PER_W = TOTAL // NW
CH = 128
HCH = CH // 2
NCH = PER_W // CH
NCHP = 56

LT = 8
NLC = L // LT
BP = B // 2
DP = 2 * EMB
HP = 2 * H


def _sc_gather_body(table_hbm, idx_hbm, out_hbm, idx_v, rows_a, rows_b, sem):
    wid = lax.axis_index("s") * 2 + lax.axis_index("c")
    pltpu.sync_copy(idx_hbm.at[wid], idx_v)

    def body(j, _):
        ca = pltpu.async_copy(table_hbm.at[idx_v.at[j, pl.ds(0, HCH)]],
                              rows_a, sem)
        cb = pltpu.async_copy(table_hbm.at[idx_v.at[j, pl.ds(HCH, HCH)]],
                              rows_b, sem)
        ca.wait()
        cb.wait()
        base = (wid * NCH + j) * HCH
        pltpu.sync_copy(rows_a,
                        out_hbm.at[pl.ds(base, HCH), pl.ds(0, EMB)])
        pltpu.sync_copy(rows_b,
                        out_hbm.at[pl.ds(base, HCH), pl.ds(EMB, EMB)])
        return 0

    lax.fori_loop(0, NCH, body, 0)


@functools.cache
def _sc_gather():
    return pl.kernel(
        _sc_gather_body,
        out_type=jax.ShapeDtypeStruct((TOTAL // 2, DP), jnp.float32),
        mesh=plsc.VectorSubcoreMesh(core_axis_name="c", subcore_axis_name="s"),
        scratch_types=[
            pltpu.VMEM((NCHP, CH), jnp.int32),
            pltpu.VMEM((HCH, EMB), jnp.float32),
            pltpu.VMEM((HCH, EMB), jnp.float32),
            pltpu.SemaphoreType.DMA,
        ],
        compiler_params=pltpu.CompilerParams(use_tc_tiling_on_sc=False),
    )


def _rnn_body(e_ref, w1_ref, wh1_ref, w2_ref, wh2_ref, fct_ref,
              b1_ref, b2_ref, fcb_ref, out_ref, h1_ref, h2_ref, h1buf_ref):
    lc = pl.program_id(0)

    @pl.when(lc == 0)
    def _():
        h1_ref[...] = jnp.zeros_like(h1_ref)
        h2_ref[...] = jnp.zeros_like(h2_ref)

    xp1 = jnp.dot(e_ref[...], w1_ref[...], preferred_element_type=jnp.float32)
    xp1 = xp1 + b1_ref[...]

    h1 = h1_ref[...]
    for t in range(LT):
        h1 = jnp.tanh(
            xp1[t * BP:(t + 1) * BP]
            + jnp.dot(h1, wh1_ref[...], preferred_element_type=jnp.float32))
        h1buf_ref[t * BP:(t + 1) * BP] = h1
    h1_ref[...] = h1

    xp2 = jnp.dot(h1buf_ref[...], w2_ref[...],
                  preferred_element_type=jnp.float32)
    xp2 = xp2 + b2_ref[...]

    h2 = h2_ref[...]
    for t in range(LT):
        h2 = jnp.tanh(
            xp2[t * BP:(t + 1) * BP]
            + jnp.dot(h2, wh2_ref[...], preferred_element_type=jnp.float32))
    h2_ref[...] = h2

    @pl.when(lc == NLC - 1)
    def _():
        out_ref[...] = (
            jnp.dot(h2, fct_ref[...], preferred_element_type=jnp.float32)
            + fcb_ref[...])


_rnn_call = pl.pallas_call(
    _rnn_body,
    grid=(NLC,),
    in_specs=[
        pl.BlockSpec((LT * BP, DP), lambda l: (l, 0)),
        pl.BlockSpec((DP, HP), lambda l: (0, 0)),
        pl.BlockSpec((HP, HP), lambda l: (0, 0)),
        pl.BlockSpec((HP, HP), lambda l: (0, 0)),
        pl.BlockSpec((HP, HP), lambda l: (0, 0)),
        pl.BlockSpec((HP, 2 * NCLS), lambda l: (0, 0)),
        pl.BlockSpec((1, HP), lambda l: (0, 0)),
        pl.BlockSpec((1, HP), lambda l: (0, 0)),
        pl.BlockSpec((1, 2 * NCLS), lambda l: (0, 0)),
    ],
    out_specs=pl.BlockSpec((BP, 2 * NCLS), lambda l: (0, 0)),
    out_shape=jax.ShapeDtypeStruct((BP, 2 * NCLS), jnp.float32),
    scratch_shapes=[
        pltpu.VMEM((BP, HP), jnp.float32),
        pltpu.VMEM((BP, HP), jnp.float32),
        pltpu.VMEM((LT * BP, HP), jnp.float32),
    ],
    compiler_params=pltpu.CompilerParams(
        dimension_semantics=("arbitrary",)),
)


def _blkdiag(a):
    m, n = a.shape
    z = jnp.zeros((m, n), a.dtype)
    return jnp.concatenate(
        [jnp.concatenate([a, z], axis=1), jnp.concatenate([z, a], axis=1)],
        axis=0)


def kernel(x, emb, w_ih1, w_hh1, b_ih1, b_hh1,
           w_ih2, w_hh2, b_ih2, b_hh2, fc_w, fc_b):
    idxp = (x.T.astype(jnp.int32)
            .reshape(NW, NCH, HCH, 2)
            .transpose(0, 1, 3, 2)
            .reshape(NW, NCH, CH))
    idxp = jnp.pad(idxp, ((0, 0), (0, NCHP - NCH), (0, 0)))
    e = _sc_gather()(emb, idxp)

    b1 = jnp.concatenate([b_ih1 + b_hh1] * 2)[None, :]
    b2 = jnp.concatenate([b_ih2 + b_hh2] * 2)[None, :]
    fcb = jnp.concatenate([fc_b] * 2)[None, :]
    out = _rnn_call(
        e,
        _blkdiag(w_ih1.T), _blkdiag(w_hh1.T),
        _blkdiag(w_ih2.T), _blkdiag(w_hh2.T), _blkdiag(fc_w.T),
        b1, b2, fcb)
    return out.reshape(B, NCLS)
